# Initial kernel scaffold; baseline (speedup 1.0000x reference)
import jax
import jax.numpy as jnp
from jax.experimental import pallas as pl


def kernel(point_1, point_2, point_feat_1, point_feat_2, W1, g1, b1, W2, g2, b2):
    raise NotImplementedError("write your pallas kernel here")


# speedup vs baseline: 21.2781x; 21.2781x over previous
import functools

import jax
import jax.numpy as jnp
from jax.experimental import pallas as pl

B, N, S = 8, 4096, 1024
C1, C2 = 256, 512
O1, O2 = 256, 256
EPS_BN = 1e-5
NB = 512
NBLK = N // NB
BIG = 3.0e38


def _g_kernel(f2_ref, w1b_ref, g_ref):
    g_ref[0] = jax.lax.dot_general(
        f2_ref[0], w1b_ref[...], (((1,), (1,)), ((), ())),
        preferred_element_type=jnp.float32)


def _main_kernel(p1_ref, p2_ref, f1_ref, g_ref, w1a_ref, h1_ref, st_ref):
    p1 = p1_ref[0]
    p2 = p2_ref[0]
    d = jnp.zeros((NB, S), jnp.float32)
    for j in range(3):
        t = p1[:, j][:, None] - p2[:, j][None, :]
        d = d + t * t
    iota = jax.lax.broadcasted_iota(jnp.int32, (NB, S), 1)
    onehot = jnp.zeros((NB, S), jnp.float32)
    recips = []
    sels = []
    for _ in range(3):
        m = jnp.min(d, axis=1, keepdims=True)
        idx = jnp.min(jnp.where(d == m, iota, S), axis=1, keepdims=True)
        sel = (iota == idx)
        sels.append(sel)
        recips.append(1.0 / (m + 1e-8))
        d = jnp.where(sel, BIG, d)
    wsum = recips[0] + recips[1] + recips[2]
    for k in range(3):
        onehot = onehot + jnp.where(sels[k], (recips[k] / wsum), 0.0)
    interp = jax.lax.dot_general(
        onehot, g_ref[0], (((1,), (0,)), ((), ())),
        preferred_element_type=jnp.float32)
    h1 = interp + jax.lax.dot_general(
        f1_ref[0], w1a_ref[...], (((1,), (1,)), ((), ())),
        preferred_element_type=jnp.float32)
    h1_ref[0] = h1
    part = jnp.concatenate([jnp.sum(h1, axis=0)[None, :],
                            jnp.sum(h1 * h1, axis=0)[None, :]], axis=0)
    first = (pl.program_id(0) == 0) & (pl.program_id(1) == 0)

    @pl.when(first)
    def _():
        st_ref[...] = part

    @pl.when(jnp.logical_not(first))
    def _():
        st_ref[...] += part


def _layer2_kernel(h1_ref, sc_ref, sh_ref, w2_ref, h2_ref, st_ref):
    x = jnp.maximum(h1_ref[0] * sc_ref[0][None, :] + sh_ref[0][None, :], 0.0)
    h2 = jax.lax.dot_general(
        x, w2_ref[...], (((1,), (1,)), ((), ())),
        preferred_element_type=jnp.float32)
    h2_ref[0] = h2
    part = jnp.concatenate([jnp.sum(h2, axis=0)[None, :],
                            jnp.sum(h2 * h2, axis=0)[None, :]], axis=0)
    first = (pl.program_id(0) == 0) & (pl.program_id(1) == 0)

    @pl.when(first)
    def _():
        st_ref[...] = part

    @pl.when(jnp.logical_not(first))
    def _():
        st_ref[...] += part


def _final_kernel(h2_ref, sc_ref, sh_ref, o_ref):
    o_ref[0] = jnp.maximum(
        h2_ref[0] * sc_ref[0][None, :] + sh_ref[0][None, :], 0.0)


def _bn_affine(st, g, b):
    cnt = float(B * N)
    mean = st[0] / cnt
    var = jnp.maximum(st[1] / cnt - mean * mean, 0.0)
    scale = g * jax.lax.rsqrt(var + EPS_BN)
    shift = b - mean * scale
    return scale[None, :], shift[None, :]


@jax.jit
def _run(point_1, point_2, point_feat_1, point_feat_2, W1, g1, b1, W2, g2, b2):
    W1a = W1[:, :C1]
    W1b = W1[:, C1:]

    G = pl.pallas_call(
        _g_kernel,
        grid=(B,),
        in_specs=[
            pl.BlockSpec((1, S, C2), lambda b: (b, 0, 0)),
            pl.BlockSpec((O1, C2), lambda b: (0, 0)),
        ],
        out_specs=pl.BlockSpec((1, S, O1), lambda b: (b, 0, 0)),
        out_shape=jax.ShapeDtypeStruct((B, S, O1), jnp.float32),
    )(point_feat_2, W1b)

    h1, st1 = pl.pallas_call(
        _main_kernel,
        grid=(B, NBLK),
        in_specs=[
            pl.BlockSpec((1, NB, 3), lambda b, i: (b, i, 0)),
            pl.BlockSpec((1, S, 3), lambda b, i: (b, 0, 0)),
            pl.BlockSpec((1, NB, C1), lambda b, i: (b, i, 0)),
            pl.BlockSpec((1, S, O1), lambda b, i: (b, 0, 0)),
            pl.BlockSpec((O1, C1), lambda b, i: (0, 0)),
        ],
        out_specs=[
            pl.BlockSpec((1, NB, O1), lambda b, i: (b, i, 0)),
            pl.BlockSpec((2, O1), lambda b, i: (0, 0)),
        ],
        out_shape=[
            jax.ShapeDtypeStruct((B, N, O1), jnp.float32),
            jax.ShapeDtypeStruct((2, O1), jnp.float32),
        ],
    )(point_1, point_2, point_feat_1, G, W1a)

    sc1, sh1 = _bn_affine(st1, g1, b1)

    h2, st2 = pl.pallas_call(
        _layer2_kernel,
        grid=(B, NBLK),
        in_specs=[
            pl.BlockSpec((1, NB, O1), lambda b, i: (b, i, 0)),
            pl.BlockSpec((1, O1), lambda b, i: (0, 0)),
            pl.BlockSpec((1, O1), lambda b, i: (0, 0)),
            pl.BlockSpec((O2, O1), lambda b, i: (0, 0)),
        ],
        out_specs=[
            pl.BlockSpec((1, NB, O2), lambda b, i: (b, i, 0)),
            pl.BlockSpec((2, O2), lambda b, i: (0, 0)),
        ],
        out_shape=[
            jax.ShapeDtypeStruct((B, N, O2), jnp.float32),
            jax.ShapeDtypeStruct((2, O2), jnp.float32),
        ],
    )(h1, sc1, sh1, W2)

    sc2, sh2 = _bn_affine(st2, g2, b2)

    out = pl.pallas_call(
        _final_kernel,
        grid=(B, NBLK),
        in_specs=[
            pl.BlockSpec((1, NB, O2), lambda b, i: (b, i, 0)),
            pl.BlockSpec((1, O2), lambda b, i: (0, 0)),
            pl.BlockSpec((1, O2), lambda b, i: (0, 0)),
        ],
        out_specs=pl.BlockSpec((1, NB, O2), lambda b, i: (b, i, 0)),
        out_shape=jax.ShapeDtypeStruct((B, N, O2), jnp.float32),
    )(h2, sc2, sh2)

    return out


def kernel(point_1, point_2, point_feat_1, point_feat_2, W1, g1, b1, W2, g2, b2):
    return _run(point_1, point_2, point_feat_1, point_feat_2,
                W1, g1, b1, W2, g2, b2)


# speedup vs baseline: 23.3377x; 1.0968x over previous
import jax
import jax.numpy as jnp
from jax.experimental import pallas as pl

B, N, S = 8, 4096, 1024
C1, C2 = 256, 512
O1, O2 = 256, 256
EPS_BN = 1e-5
NB = 512
NBLK = N // NB
IMAX = jnp.iinfo(jnp.int32).max
CNT = float(B * N)


def _g_kernel(f2_ref, w1b_ref, g_ref):
    g_ref[0] = jax.lax.dot_general(
        f2_ref[0], w1b_ref[...], (((1,), (1,)), ((), ())),
        preferred_element_type=jnp.float32)


def _main_kernel(p1_ref, p2_ref, f1_ref, g_ref, w1a_ref, h1_ref, st_ref):
    p1 = p1_ref[0]
    p2 = p2_ref[0]
    d = jnp.zeros((NB, S), jnp.float32)
    for j in range(3):
        t = p1[:, j][:, None] - p2[:, j][None, :]
        d = d + t * t
    iota = jax.lax.broadcasted_iota(jnp.int32, (NB, S), 1)
    key = (jax.lax.bitcast_convert_type(d, jnp.int32) & -1024) | iota
    m1 = jnp.full((NB, 128), IMAX, jnp.int32)
    m2 = m1
    m3 = m1
    for c in range(8):
        x = key[:, c * 128:(c + 1) * 128]
        hi = jnp.maximum(m1, x)
        m1 = jnp.minimum(m1, x)
        hi2 = jnp.maximum(m2, hi)
        m2 = jnp.minimum(m2, hi)
        m3 = jnp.minimum(m3, hi2)
    ks = []
    for r in range(3):
        k = jnp.min(m1, axis=1, keepdims=True)
        ks.append(k)
        if r < 2:
            sel = (m1 == k)
            m1 = jnp.where(sel, m2, m1)
            m2 = jnp.where(sel, m3, m2)
    recips = []
    idxs = []
    for k in ks:
        dk = jax.lax.bitcast_convert_type(k & -1024, jnp.float32)
        recips.append(1.0 / (dk + 1e-8))
        idxs.append(k & 1023)
    wsum = recips[0] + recips[1] + recips[2]
    onehot = jnp.where(iota == idxs[0], recips[0] / wsum, 0.0)
    onehot = jnp.where(iota == idxs[1], recips[1] / wsum, onehot)
    onehot = jnp.where(iota == idxs[2], recips[2] / wsum, onehot)
    interp = jax.lax.dot_general(
        onehot, g_ref[0], (((1,), (0,)), ((), ())),
        preferred_element_type=jnp.float32)
    h1 = interp + jax.lax.dot_general(
        f1_ref[0], w1a_ref[...], (((1,), (1,)), ((), ())),
        preferred_element_type=jnp.float32)
    h1_ref[0] = h1
    part = jnp.concatenate([jnp.sum(h1, axis=0)[None, :],
                            jnp.sum(h1 * h1, axis=0)[None, :]], axis=0)
    first = (pl.program_id(0) == 0) & (pl.program_id(1) == 0)

    @pl.when(first)
    def _():
        st_ref[...] = part

    @pl.when(jnp.logical_not(first))
    def _():
        st_ref[...] += part


def _bn_affine_in_kernel(st_ref, g_ref, b_ref):
    mean = st_ref[0:1, :] * (1.0 / CNT)
    var = jnp.maximum(st_ref[1:2, :] * (1.0 / CNT) - mean * mean, 0.0)
    scale = g_ref[...] * jax.lax.rsqrt(var + EPS_BN)
    shift = b_ref[...] - mean * scale
    return scale, shift


def _layer2_kernel(h1_ref, st1_ref, g_ref, b_ref, w2_ref, h2_ref, st_ref):
    scale, shift = _bn_affine_in_kernel(st1_ref, g_ref, b_ref)
    x = jnp.maximum(h1_ref[0] * scale + shift, 0.0)
    h2 = jax.lax.dot_general(
        x, w2_ref[...], (((1,), (1,)), ((), ())),
        preferred_element_type=jnp.float32)
    h2_ref[0] = h2
    part = jnp.concatenate([jnp.sum(h2, axis=0)[None, :],
                            jnp.sum(h2 * h2, axis=0)[None, :]], axis=0)
    first = (pl.program_id(0) == 0) & (pl.program_id(1) == 0)

    @pl.when(first)
    def _():
        st_ref[...] = part

    @pl.when(jnp.logical_not(first))
    def _():
        st_ref[...] += part


def _final_kernel(h2_ref, st2_ref, g_ref, b_ref, o_ref):
    scale, shift = _bn_affine_in_kernel(st2_ref, g_ref, b_ref)
    o_ref[0] = jnp.maximum(h2_ref[0] * scale + shift, 0.0)


@jax.jit
def _run(point_1, point_2, point_feat_1, point_feat_2, W1, g1, b1, W2, g2, b2):
    W1a = W1[:, :C1]
    W1b = W1[:, C1:]
    g1r = g1.reshape(1, O1)
    b1r = b1.reshape(1, O1)
    g2r = g2.reshape(1, O2)
    b2r = b2.reshape(1, O2)

    G = pl.pallas_call(
        _g_kernel,
        grid=(B,),
        in_specs=[
            pl.BlockSpec((1, S, C2), lambda b: (b, 0, 0)),
            pl.BlockSpec((O1, C2), lambda b: (0, 0)),
        ],
        out_specs=pl.BlockSpec((1, S, O1), lambda b: (b, 0, 0)),
        out_shape=jax.ShapeDtypeStruct((B, S, O1), jnp.float32),
    )(point_feat_2, W1b)

    h1, st1 = pl.pallas_call(
        _main_kernel,
        grid=(B, NBLK),
        in_specs=[
            pl.BlockSpec((1, NB, 3), lambda b, i: (b, i, 0)),
            pl.BlockSpec((1, S, 3), lambda b, i: (b, 0, 0)),
            pl.BlockSpec((1, NB, C1), lambda b, i: (b, i, 0)),
            pl.BlockSpec((1, S, O1), lambda b, i: (b, 0, 0)),
            pl.BlockSpec((O1, C1), lambda b, i: (0, 0)),
        ],
        out_specs=[
            pl.BlockSpec((1, NB, O1), lambda b, i: (b, i, 0)),
            pl.BlockSpec((2, O1), lambda b, i: (0, 0)),
        ],
        out_shape=[
            jax.ShapeDtypeStruct((B, N, O1), jnp.float32),
            jax.ShapeDtypeStruct((2, O1), jnp.float32),
        ],
    )(point_1, point_2, point_feat_1, G, W1a)

    h2, st2 = pl.pallas_call(
        _layer2_kernel,
        grid=(B, NBLK),
        in_specs=[
            pl.BlockSpec((1, NB, O1), lambda b, i: (b, i, 0)),
            pl.BlockSpec((2, O1), lambda b, i: (0, 0)),
            pl.BlockSpec((1, O1), lambda b, i: (0, 0)),
            pl.BlockSpec((1, O1), lambda b, i: (0, 0)),
            pl.BlockSpec((O2, O1), lambda b, i: (0, 0)),
        ],
        out_specs=[
            pl.BlockSpec((1, NB, O2), lambda b, i: (b, i, 0)),
            pl.BlockSpec((2, O2), lambda b, i: (0, 0)),
        ],
        out_shape=[
            jax.ShapeDtypeStruct((B, N, O2), jnp.float32),
            jax.ShapeDtypeStruct((2, O2), jnp.float32),
        ],
    )(h1, st1, g1r, b1r, W2)

    out = pl.pallas_call(
        _final_kernel,
        grid=(B, NBLK),
        in_specs=[
            pl.BlockSpec((1, NB, O2), lambda b, i: (b, i, 0)),
            pl.BlockSpec((2, O2), lambda b, i: (0, 0)),
            pl.BlockSpec((1, O2), lambda b, i: (0, 0)),
            pl.BlockSpec((1, O2), lambda b, i: (0, 0)),
        ],
        out_specs=pl.BlockSpec((1, NB, O2), lambda b, i: (b, i, 0)),
        out_shape=jax.ShapeDtypeStruct((B, N, O2), jnp.float32),
    )(h2, st2, g2r, b2r)

    return out


def kernel(point_1, point_2, point_feat_1, point_feat_2, W1, g1, b1, W2, g2, b2):
    return _run(point_1, point_2, point_feat_1, point_feat_2,
                W1, g1, b1, W2, g2, b2)


# speedup vs baseline: 25.0286x; 1.0725x over previous
import jax
import jax.numpy as jnp
from jax.experimental import pallas as pl

B, N, S = 8, 4096, 1024
C1, C2 = 256, 512
O1, O2 = 256, 256
EPS_BN = 1e-5
NB = 512
NBLK = N // NB
IMAX = jnp.iinfo(jnp.int32).max
CNT = float(B * N)


def _g_kernel(f2_ref, w1b_ref, g_ref):
    g_ref[0] = jax.lax.dot_general(
        f2_ref[0], w1b_ref[...], (((1,), (1,)), ((), ())),
        preferred_element_type=jnp.float32)


def _main_kernel(p1_ref, p2_ref, f1_ref, g_ref, w1a_ref, h1_ref, st_ref):
    p1 = p1_ref[0]
    p2 = p2_ref[0]
    d = jnp.zeros((NB, S), jnp.float32)
    for j in range(3):
        t = p1[:, j][:, None] - p2[:, j][None, :]
        d = d + t * t
    iota = jax.lax.broadcasted_iota(jnp.int32, (NB, S), 1)
    key = jax.lax.bitcast_convert_type(
        (jax.lax.bitcast_convert_type(d, jnp.int32) & -1024) | iota,
        jnp.float32)
    m1 = jnp.full((NB, 128), 3.0e38, jnp.float32)
    m2 = m1
    m3 = m1
    for c in range(8):
        x = key[:, c * 128:(c + 1) * 128]
        hi = jnp.maximum(m1, x)
        m1 = jnp.minimum(m1, x)
        hi2 = jnp.maximum(m2, hi)
        m2 = jnp.minimum(m2, hi)
        m3 = jnp.minimum(m3, hi2)
    ks = []
    for r in range(3):
        k = jnp.min(m1, axis=1, keepdims=True)
        ks.append(k)
        if r < 2:
            sel = (m1 == k)
            m1 = jnp.where(sel, m2, m1)
            m2 = jnp.where(sel, m3, m2)
    recips = []
    for k in ks:
        ki = jax.lax.bitcast_convert_type(k, jnp.int32)
        dk = jax.lax.bitcast_convert_type(ki & -1024, jnp.float32)
        recips.append(1.0 / (dk + 1e-8))
    wsum = recips[0] + recips[1] + recips[2]
    onehot = jnp.where(key == ks[0], recips[0] / wsum, 0.0)
    onehot = jnp.where(key == ks[1], recips[1] / wsum, onehot)
    onehot = jnp.where(key == ks[2], recips[2] / wsum, onehot)
    interp = jax.lax.dot_general(
        onehot, g_ref[0], (((1,), (0,)), ((), ())),
        preferred_element_type=jnp.float32)
    h1 = interp + jax.lax.dot_general(
        f1_ref[0], w1a_ref[...], (((1,), (1,)), ((), ())),
        preferred_element_type=jnp.float32)
    h1_ref[0] = h1
    part = jnp.concatenate([jnp.sum(h1, axis=0)[None, :],
                            jnp.sum(h1 * h1, axis=0)[None, :]], axis=0)
    first = (pl.program_id(0) == 0) & (pl.program_id(1) == 0)

    @pl.when(first)
    def _():
        st_ref[...] = part

    @pl.when(jnp.logical_not(first))
    def _():
        st_ref[...] += part


def _bn_affine_in_kernel(st_ref, g_ref, b_ref):
    mean = st_ref[0:1, :] * (1.0 / CNT)
    var = jnp.maximum(st_ref[1:2, :] * (1.0 / CNT) - mean * mean, 0.0)
    scale = g_ref[...] * jax.lax.rsqrt(var + EPS_BN)
    shift = b_ref[...] - mean * scale
    return scale, shift


def _layer2_kernel(h1_ref, st1_ref, g_ref, b_ref, w2_ref, h2_ref, st_ref):
    scale, shift = _bn_affine_in_kernel(st1_ref, g_ref, b_ref)
    x = jnp.maximum(h1_ref[0] * scale + shift, 0.0)
    h2 = jax.lax.dot_general(
        x, w2_ref[...], (((1,), (1,)), ((), ())),
        preferred_element_type=jnp.float32)
    h2_ref[0] = h2
    part = jnp.concatenate([jnp.sum(h2, axis=0)[None, :],
                            jnp.sum(h2 * h2, axis=0)[None, :]], axis=0)
    first = (pl.program_id(0) == 0) & (pl.program_id(1) == 0)

    @pl.when(first)
    def _():
        st_ref[...] = part

    @pl.when(jnp.logical_not(first))
    def _():
        st_ref[...] += part


def _final_kernel(h2_ref, st2_ref, g_ref, b_ref, o_ref):
    scale, shift = _bn_affine_in_kernel(st2_ref, g_ref, b_ref)
    o_ref[0] = jnp.maximum(h2_ref[0] * scale + shift, 0.0)


@jax.jit
def _run(point_1, point_2, point_feat_1, point_feat_2, W1, g1, b1, W2, g2, b2):
    W1a = W1[:, :C1]
    W1b = W1[:, C1:]
    g1r = g1.reshape(1, O1)
    b1r = b1.reshape(1, O1)
    g2r = g2.reshape(1, O2)
    b2r = b2.reshape(1, O2)

    G = pl.pallas_call(
        _g_kernel,
        grid=(B,),
        in_specs=[
            pl.BlockSpec((1, S, C2), lambda b: (b, 0, 0)),
            pl.BlockSpec((O1, C2), lambda b: (0, 0)),
        ],
        out_specs=pl.BlockSpec((1, S, O1), lambda b: (b, 0, 0)),
        out_shape=jax.ShapeDtypeStruct((B, S, O1), jnp.float32),
    )(point_feat_2, W1b)

    h1, st1 = pl.pallas_call(
        _main_kernel,
        grid=(B, NBLK),
        in_specs=[
            pl.BlockSpec((1, NB, 3), lambda b, i: (b, i, 0)),
            pl.BlockSpec((1, S, 3), lambda b, i: (b, 0, 0)),
            pl.BlockSpec((1, NB, C1), lambda b, i: (b, i, 0)),
            pl.BlockSpec((1, S, O1), lambda b, i: (b, 0, 0)),
            pl.BlockSpec((O1, C1), lambda b, i: (0, 0)),
        ],
        out_specs=[
            pl.BlockSpec((1, NB, O1), lambda b, i: (b, i, 0)),
            pl.BlockSpec((2, O1), lambda b, i: (0, 0)),
        ],
        out_shape=[
            jax.ShapeDtypeStruct((B, N, O1), jnp.float32),
            jax.ShapeDtypeStruct((2, O1), jnp.float32),
        ],
    )(point_1, point_2, point_feat_1, G, W1a)

    h2, st2 = pl.pallas_call(
        _layer2_kernel,
        grid=(B, NBLK),
        in_specs=[
            pl.BlockSpec((1, NB, O1), lambda b, i: (b, i, 0)),
            pl.BlockSpec((2, O1), lambda b, i: (0, 0)),
            pl.BlockSpec((1, O1), lambda b, i: (0, 0)),
            pl.BlockSpec((1, O1), lambda b, i: (0, 0)),
            pl.BlockSpec((O2, O1), lambda b, i: (0, 0)),
        ],
        out_specs=[
            pl.BlockSpec((1, NB, O2), lambda b, i: (b, i, 0)),
            pl.BlockSpec((2, O2), lambda b, i: (0, 0)),
        ],
        out_shape=[
            jax.ShapeDtypeStruct((B, N, O2), jnp.float32),
            jax.ShapeDtypeStruct((2, O2), jnp.float32),
        ],
    )(h1, st1, g1r, b1r, W2)

    out = pl.pallas_call(
        _final_kernel,
        grid=(B, NBLK),
        in_specs=[
            pl.BlockSpec((1, NB, O2), lambda b, i: (b, i, 0)),
            pl.BlockSpec((2, O2), lambda b, i: (0, 0)),
            pl.BlockSpec((1, O2), lambda b, i: (0, 0)),
            pl.BlockSpec((1, O2), lambda b, i: (0, 0)),
        ],
        out_specs=pl.BlockSpec((1, NB, O2), lambda b, i: (b, i, 0)),
        out_shape=jax.ShapeDtypeStruct((B, N, O2), jnp.float32),
    )(h2, st2, g2r, b2r)

    return out


def kernel(point_1, point_2, point_feat_1, point_feat_2, W1, g1, b1, W2, g2, b2):
    return _run(point_1, point_2, point_feat_1, point_feat_2,
                W1, g1, b1, W2, g2, b2)
